# baseline (device time: 45283 ns/iter reference)
import jax
import jax.numpy as jnp
from jax import lax
from jax.experimental import pallas as pl
from jax.experimental.pallas import tpu as pltpu

N_DEV = 4
E_PER = 4
CAP = 204
N_EXP = N_DEV * E_PER
HALF = E_PER // 2


def kernel(x, router_W, route_idx, expert_W):
    n_tok, d_model = x.shape
    e_per, _, d_hid = expert_W.shape

    def body(x_ref, rw_ref, idx_ref, ew_ref, out_ref,
             ew_bf, b_left, b_right, b_diag,
             rb_left, rb_right, rb_diag,
             send_sems, recv_sems):
        my = lax.axis_index("i")
        left = (my - 1) % N_DEV
        right = (my + 1) % N_DEV

        barrier_sem = pltpu.get_barrier_semaphore()
        for nbr in [left, right]:
            pl.semaphore_signal(
                barrier_sem, inc=1,
                device_id=(nbr,), device_id_type=pl.DeviceIdType.MESH,
            )
        pl.semaphore_wait(barrier_sem, 2)

        ew_bf[...] = ew_ref[...].astype(jnp.bfloat16)

        def rdma(i, src, dst, dev):
            return pltpu.make_async_remote_copy(
                src_ref=src, dst_ref=dst,
                send_sem=send_sems.at[i], recv_sem=recv_sems.at[i],
                device_id=(dev,), device_id_type=pl.DeviceIdType.MESH,
            )

        p1_er = rdma(0, ew_bf, b_left, right)
        p1_el = rdma(1, ew_bf, b_right, left)
        p1_rr = rdma(2, idx_ref, rb_left, right)
        p1_rl = rdma(3, idx_ref, rb_right, left)
        p1_er.start()
        p1_el.start()
        p1_rr.start()
        p1_rl.start()

        xsb = x_ref[...].astype(jnp.bfloat16)
        routes = idx_ref[...]

        def chunk_acc(w_ref, origin, acc):
            xm = jnp.concatenate(
                [
                    xsb * (routes == origin * E_PER + k).astype(jnp.bfloat16)
                    for k in range(E_PER)
                ],
                axis=1,
            )
            w = w_ref[...].reshape(E_PER * d_model, d_hid)
            return acc + jnp.dot(xm, w, preferred_element_type=jnp.float32)

        acc = jnp.zeros((n_tok, d_hid), jnp.float32)
        acc = chunk_acc(ew_bf, my, acc)

        p1_rr.wait_recv()
        p1_rl.wait_recv()
        p2_r = rdma(4, rb_left, rb_diag, right)
        p2_r.start()

        eids = lax.broadcasted_iota(jnp.int32, (n_tok, N_EXP), 1)
        onehot = (routes == eids).astype(jnp.float32)
        row = lax.broadcasted_iota(jnp.int32, (n_tok, n_tok), 0)
        col = lax.broadcasted_iota(jnp.int32, (n_tok, n_tok), 1)
        tri = (col < row).astype(jnp.float32)
        cumex = jnp.dot(tri, onehot, preferred_element_type=jnp.float32)

        p2_r.wait_recv()
        prev = jnp.zeros((1, N_EXP), jnp.float32)
        for rb, delta in ((rb_left, 3), (rb_right, 1), (rb_diag, 2)):
            origin = (my + delta) % N_DEV
            oh = (rb[...] == eids).astype(jnp.float32)
            cnt = jnp.sum(oh, axis=0, keepdims=True)
            prev = prev + jnp.where(origin < my, cnt, 0.0)
        ranks = prev + cumex
        keep = jnp.sum(
            onehot * (ranks < float(CAP)).astype(jnp.float32),
            axis=1, keepdims=True,
        )

        p1_er.wait_recv()
        p2_er = rdma(5, b_left.at[HALF:E_PER], b_diag.at[HALF:E_PER], right)
        p2_er.start()
        p1_el.wait_recv()
        p2_el = rdma(6, b_right.at[0:HALF], b_diag.at[0:HALF], left)
        p2_el.start()

        acc = chunk_acc(b_left, (my + 3) % N_DEV, acc)
        acc = chunk_acc(b_right, (my + 1) % N_DEV, acc)

        p2_er.wait_recv()
        p2_el.wait_recv()
        acc = chunk_acc(b_diag, (my + 2) % N_DEV, acc)

        out_ref[...] = acc * keep

        for d in (p1_er, p1_el, p1_rr, p1_rl, p2_r, p2_er, p2_el):
            d.wait_send()

    return pl.pallas_call(
        body,
        out_shape=jax.ShapeDtypeStruct((n_tok, d_hid), jnp.float32),
        in_specs=[
            pl.BlockSpec(memory_space=pltpu.VMEM),
            pl.BlockSpec(memory_space=pltpu.VMEM),
            pl.BlockSpec(memory_space=pltpu.VMEM),
            pl.BlockSpec(memory_space=pltpu.VMEM),
        ],
        out_specs=pl.BlockSpec(memory_space=pltpu.VMEM),
        scratch_shapes=[
            pltpu.VMEM((e_per, d_model, d_hid), jnp.bfloat16),
            pltpu.VMEM((e_per, d_model, d_hid), jnp.bfloat16),
            pltpu.VMEM((e_per, d_model, d_hid), jnp.bfloat16),
            pltpu.VMEM((e_per, d_model, d_hid), jnp.bfloat16),
            pltpu.VMEM((n_tok, 1), jnp.int32),
            pltpu.VMEM((n_tok, 1), jnp.int32),
            pltpu.VMEM((n_tok, 1), jnp.int32),
            pltpu.SemaphoreType.DMA((7,)),
            pltpu.SemaphoreType.DMA((7,)),
        ],
        compiler_params=pltpu.CompilerParams(collective_id=0),
    )(x, router_W, route_idx, expert_W)


# device time: 30786 ns/iter; 1.4709x vs baseline; 1.4709x over previous
import jax
import jax.numpy as jnp
from jax import lax
from jax.experimental import pallas as pl
from jax.experimental.pallas import tpu as pltpu

N_DEV = 4
E_PER = 4
CAP = 204
N_EXP = N_DEV * E_PER
HALF = E_PER // 2


def kernel(x, router_W, route_idx, expert_W):
    n_tok, d_model = x.shape
    e_per, _, d_hid = expert_W.shape

    def body(x_ref, rw_ref, idx_ref, ew_ref, out_ref,
             ew_bf, b_left, b_right, b_diag,
             cb_self, cb_left, cb_right, cb_diag,
             send_sems, recv_sems):
        my = lax.axis_index("i")
        left = (my - 1) % N_DEV
        right = (my + 1) % N_DEV

        barrier_sem = pltpu.get_barrier_semaphore()
        for nbr in [left, right]:
            pl.semaphore_signal(
                barrier_sem, inc=1,
                device_id=(nbr,), device_id_type=pl.DeviceIdType.MESH,
            )
        pl.semaphore_wait(barrier_sem, 2)

        ew_bf[...] = ew_ref[...].astype(jnp.bfloat16)
        routes = idx_ref[...]
        eids = lax.broadcasted_iota(jnp.int32, (n_tok, N_EXP), 1)
        onehot = (routes == eids).astype(jnp.float32)
        cb_self[...] = jnp.sum(onehot, axis=0, keepdims=True)

        def rdma(i, src, dst, dev):
            return pltpu.make_async_remote_copy(
                src_ref=src, dst_ref=dst,
                send_sem=send_sems.at[i], recv_sem=recv_sems.at[i],
                device_id=(dev,), device_id_type=pl.DeviceIdType.MESH,
            )

        p1_er = rdma(0, ew_bf, b_left, right)
        p1_el = rdma(1, ew_bf, b_right, left)
        p1_cr = rdma(2, cb_self, cb_left, right)
        p1_cl = rdma(3, cb_self, cb_right, left)
        p1_er.start()
        p1_el.start()
        p1_cr.start()
        p1_cl.start()

        xsb = x_ref[...].astype(jnp.bfloat16)

        def chunk_acc(w_ref, origin, acc):
            xm = jnp.concatenate(
                [
                    xsb * (routes == origin * E_PER + k).astype(jnp.bfloat16)
                    for k in range(E_PER)
                ],
                axis=1,
            )
            w = w_ref[...].reshape(E_PER * d_model, d_hid)
            return acc + jnp.dot(xm, w, preferred_element_type=jnp.float32)

        acc = jnp.zeros((n_tok, d_hid), jnp.float32)
        acc = chunk_acc(ew_bf, my, acc)

        row = lax.broadcasted_iota(jnp.int32, (n_tok, n_tok), 0)
        col = lax.broadcasted_iota(jnp.int32, (n_tok, n_tok), 1)
        tri = (col < row).astype(jnp.float32)
        cumex = jnp.dot(tri, onehot, preferred_element_type=jnp.float32)

        p1_cr.wait_recv()
        p1_cl.wait_recv()
        p2_c = rdma(4, cb_left, cb_diag, right)
        p2_c.start()

        p1_er.wait_recv()
        p2_er = rdma(5, b_left.at[HALF:E_PER], b_diag.at[HALF:E_PER], right)
        p2_er.start()
        p1_el.wait_recv()
        p2_el = rdma(6, b_right.at[0:HALF], b_diag.at[0:HALF], left)
        p2_el.start()

        p2_c.wait_recv()
        prev = jnp.zeros((1, N_EXP), jnp.float32)
        for cb, delta in ((cb_left, 3), (cb_right, 1), (cb_diag, 2)):
            origin = (my + delta) % N_DEV
            prev = prev + jnp.where(origin < my, cb[...], 0.0)
        ranks = prev + cumex
        keep = jnp.sum(
            onehot * (ranks < float(CAP)).astype(jnp.float32),
            axis=1, keepdims=True,
        )

        acc = chunk_acc(b_left, (my + 3) % N_DEV, acc)
        acc = chunk_acc(b_right, (my + 1) % N_DEV, acc)

        p2_er.wait_recv()
        p2_el.wait_recv()
        acc = chunk_acc(b_diag, (my + 2) % N_DEV, acc)

        out_ref[...] = acc * keep

        for d in (p1_er, p1_el, p1_cr, p1_cl, p2_c, p2_er, p2_el):
            d.wait_send()

    return pl.pallas_call(
        body,
        out_shape=jax.ShapeDtypeStruct((n_tok, d_hid), jnp.float32),
        in_specs=[
            pl.BlockSpec(memory_space=pltpu.VMEM),
            pl.BlockSpec(memory_space=pltpu.VMEM),
            pl.BlockSpec(memory_space=pltpu.VMEM),
            pl.BlockSpec(memory_space=pltpu.VMEM),
        ],
        out_specs=pl.BlockSpec(memory_space=pltpu.VMEM),
        scratch_shapes=[
            pltpu.VMEM((e_per, d_model, d_hid), jnp.bfloat16),
            pltpu.VMEM((e_per, d_model, d_hid), jnp.bfloat16),
            pltpu.VMEM((e_per, d_model, d_hid), jnp.bfloat16),
            pltpu.VMEM((e_per, d_model, d_hid), jnp.bfloat16),
            pltpu.VMEM((1, N_EXP), jnp.float32),
            pltpu.VMEM((1, N_EXP), jnp.float32),
            pltpu.VMEM((1, N_EXP), jnp.float32),
            pltpu.VMEM((1, N_EXP), jnp.float32),
            pltpu.SemaphoreType.DMA((7,)),
            pltpu.SemaphoreType.DMA((7,)),
        ],
        compiler_params=pltpu.CompilerParams(collective_id=0),
    )(x, router_W, route_idx, expert_W)
